# baseline (speedup 1.0000x reference)
import jax
import jax.numpy as jnp
from jax.experimental import pallas as pl
from jax.experimental.pallas import tpu as pltpu

_C = 64
_K = 1024
_S = 1024
_G = 72
_BPB = 4
_CS = 256


def _vq_body(x_ref, em2_ref, g_ref, b_ref, q_ref, ind_ref, dsum_ref):
    em2 = em2_ref[...]
    g = g_ref[...]
    b = b_ref[...]
    dsum = jnp.zeros((), jnp.float32)
    aa, mm2s = [], []
    for j in range(_BPB):
        x = x_ref[j]
        aa.append(jnp.sum(x * x, axis=0, keepdims=True))
        mm2s.append(jax.lax.dot_general(
            em2, x, (((0,), (0,)), ((), ())),
            preferred_element_type=jnp.float32))
    for j in range(_BPB):
        a, mm2 = aa[j], mm2s[j]
        dist = (a + mm2) + b
        minv = jnp.min(dist, axis=0, keepdims=True)
        mask = dist == minv
        maskb = mask.astype(jnp.bfloat16)
        res = jax.lax.dot_general(
            g, maskb, (((1,), (0,)), ((), ())),
            preferred_element_type=jnp.float32)
        q_ref[j] = res[0:_C]
        ind_f = res[_C + 1:_C + 2] * 8.0 + res[_C + 2:_C + 3]
        ind_ref[j] = ind_f.astype(jnp.int32)
        dsum = dsum + jnp.sum(minv)

        cnt = res[_C:_C + 1]
        tie = jnp.max(cnt) > 1.5

        @pl.when(tie)
        def _():
            kio = jax.lax.broadcasted_iota(
                jnp.int32, (_K, _S), 0).astype(jnp.float32)
            t = jnp.where(mask, kio, jnp.float32(_K))
            ind2 = jnp.min(t, axis=0, keepdims=True)
            onehot = (t == ind2).astype(jnp.bfloat16)
            q_ref[j] = jax.lax.dot_general(
                g[0:_C], onehot, (((1,), (0,)), ((), ())),
                preferred_element_type=jnp.float32)
            ind_ref[j] = ind2.astype(jnp.int32)

    dsum_ref[...] = dsum.reshape(1, 1, 1)


def kernel(input, embedding):
    B, C, H, W = input.shape
    S = H * W
    x = input.reshape(B, C, S)
    b = jnp.sum(embedding ** 2, axis=0, keepdims=True).reshape(_K, 1)
    em2 = embedding * (-2.0)
    k = jnp.arange(_K, dtype=jnp.float32)
    g = jnp.concatenate(
        [embedding,
         jnp.ones((1, _K), jnp.float32),
         jnp.floor_divide(k, 8.0).reshape(1, _K),
         jnp.mod(k, 8.0).reshape(1, _K),
         jnp.zeros((_G - _C - 3, _K), jnp.float32)], axis=0)

    q, ind, dsum = pl.pallas_call(
        _vq_body,
        grid=(B // _BPB,),
        in_specs=[
            pl.BlockSpec((_BPB, C, S), lambda i: (i, 0, 0)),
            pl.BlockSpec((C, _K), lambda i: (0, 0)),
            pl.BlockSpec((_G, _K), lambda i: (0, 0)),
            pl.BlockSpec((_K, 1), lambda i: (0, 0)),
        ],
        out_specs=[
            pl.BlockSpec((_BPB, C, S), lambda i: (i, 0, 0)),
            pl.BlockSpec((_BPB, 1, S), lambda i: (i, 0, 0)),
            pl.BlockSpec((1, 1, 1), lambda i: (i, 0, 0)),
        ],
        out_shape=[
            jax.ShapeDtypeStruct((B, C, S), jnp.float32),
            jax.ShapeDtypeStruct((B, 1, S), jnp.int32),
            jax.ShapeDtypeStruct((B // _BPB, 1, 1), jnp.float32),
        ],
        compiler_params=pltpu.CompilerParams(
            dimension_semantics=("arbitrary",)),
    )(x, em2, g, b)

    quantize = q.reshape(B, C, H, W)
    embedding_ind = jnp.swapaxes(ind.reshape(B, H, W), 1, 2)
    diff = jnp.sum(dsum) * (0.25 / (B * S * C))
    return (quantize, diff, embedding_ind)


# speedup vs baseline: 1.0267x; 1.0267x over previous
import jax
import jax.numpy as jnp
from jax.experimental import pallas as pl
from jax.experimental.pallas import tpu as pltpu

_C = 64
_K = 1024
_S = 1024
_G = 72
_BPB = 4
_CS = 256


def _vq_body(x_ref, em2_ref, g_ref, b_ref, q_ref, ind_ref, dsum_ref):
    em2 = em2_ref[...]
    g = g_ref[...]
    b = b_ref[...]
    def _amm(j):
        x = x_ref[j]
        return (jnp.sum(x * x, axis=0, keepdims=True),
                jax.lax.dot_general(
                    em2, x, (((0,), (0,)), ((), ())),
                    preferred_element_type=jnp.float32))

    dsum = jnp.zeros((), jnp.float32)
    nxt = _amm(0)
    for j in range(_BPB):
        a, mm2 = nxt
        if j + 1 < _BPB:
            nxt = _amm(j + 1)
        dist = (a + mm2) + b
        minv = jnp.min(dist, axis=0, keepdims=True)
        mask = dist == minv
        maskb = mask.astype(jnp.bfloat16)
        res = jax.lax.dot_general(
            g, maskb, (((1,), (0,)), ((), ())),
            preferred_element_type=jnp.float32)
        q_ref[j] = res[0:_C]
        ind_f = res[_C + 1:_C + 2] * 8.0 + res[_C + 2:_C + 3]
        ind_ref[j] = ind_f.astype(jnp.int32)
        dsum = dsum + jnp.sum(minv)

        cnt = res[_C:_C + 1]
        tie = jnp.max(cnt) > 1.5

        @pl.when(tie)
        def _():
            kio = jax.lax.broadcasted_iota(
                jnp.int32, (_K, _S), 0).astype(jnp.float32)
            t = jnp.where(mask, kio, jnp.float32(_K))
            ind2 = jnp.min(t, axis=0, keepdims=True)
            onehot = (t == ind2).astype(jnp.bfloat16)
            q_ref[j] = jax.lax.dot_general(
                g[0:_C], onehot, (((1,), (0,)), ((), ())),
                preferred_element_type=jnp.float32)
            ind_ref[j] = ind2.astype(jnp.int32)

    dsum_ref[...] = dsum.reshape(1, 1, 1)


def kernel(input, embedding):
    B, C, H, W = input.shape
    S = H * W
    x = input.reshape(B, C, S)
    b = jnp.sum(embedding ** 2, axis=0, keepdims=True).reshape(_K, 1)
    em2 = embedding * (-2.0)
    k = jnp.arange(_K, dtype=jnp.float32)
    g = jnp.concatenate(
        [embedding,
         jnp.ones((1, _K), jnp.float32),
         jnp.floor_divide(k, 8.0).reshape(1, _K),
         jnp.mod(k, 8.0).reshape(1, _K),
         jnp.zeros((_G - _C - 3, _K), jnp.float32)], axis=0)

    q, ind, dsum = pl.pallas_call(
        _vq_body,
        grid=(B // _BPB,),
        in_specs=[
            pl.BlockSpec((_BPB, C, S), lambda i: (i, 0, 0)),
            pl.BlockSpec((C, _K), lambda i: (0, 0)),
            pl.BlockSpec((_G, _K), lambda i: (0, 0)),
            pl.BlockSpec((_K, 1), lambda i: (0, 0)),
        ],
        out_specs=[
            pl.BlockSpec((_BPB, C, S), lambda i: (i, 0, 0)),
            pl.BlockSpec((_BPB, 1, S), lambda i: (i, 0, 0)),
            pl.BlockSpec((1, 1, 1), lambda i: (i, 0, 0)),
        ],
        out_shape=[
            jax.ShapeDtypeStruct((B, C, S), jnp.float32),
            jax.ShapeDtypeStruct((B, 1, S), jnp.int32),
            jax.ShapeDtypeStruct((B // _BPB, 1, 1), jnp.float32),
        ],
        compiler_params=pltpu.CompilerParams(
            dimension_semantics=("arbitrary",)),
    )(x, em2, g, b)

    quantize = q.reshape(B, C, H, W)
    embedding_ind = jnp.swapaxes(ind.reshape(B, H, W), 1, 2)
    diff = jnp.sum(dsum) * (0.25 / (B * S * C))
    return (quantize, diff, embedding_ind)


# speedup vs baseline: 1.0536x; 1.0261x over previous
import jax
import jax.numpy as jnp
from jax.experimental import pallas as pl
from jax.experimental.pallas import tpu as pltpu

_C = 64
_K = 1024
_S = 1024
_G = 72
_BPB = 4
_CS = 256


def _vq_body(x_ref, em2_ref, g_ref, b_ref, q_ref, ind_ref, dsum_ref):
    em2 = em2_ref[...]
    g = g_ref[...]
    b = b_ref[...]
    dsum = jnp.zeros((), jnp.float32)
    for j in range(_BPB):
        x = x_ref[j]
        a = jnp.sum(x * x, axis=0, keepdims=True)
        mm2 = jax.lax.dot_general(
            em2, x, (((0,), (0,)), ((), ())),
            preferred_element_type=jnp.float32)
        dist = (a + mm2) + b
        minv = jnp.min(dist, axis=0, keepdims=True)
        ind = jnp.argmin(dist, axis=0).reshape(1, _S)
        kio = jax.lax.broadcasted_iota(jnp.int32, (_K, _S), 0)
        onehot = (kio == ind).astype(jnp.bfloat16)
        res = jax.lax.dot_general(
            g, onehot, (((1,), (0,)), ((), ())),
            preferred_element_type=jnp.float32)
        q_ref[j] = res[0:_C]
        ind_ref[j] = ind
        dsum = dsum + jnp.sum(minv)

    dsum_ref[...] = dsum.reshape(1, 1, 1)


def kernel(input, embedding):
    B, C, H, W = input.shape
    S = H * W
    x = input.reshape(B, C, S)
    b = jnp.sum(embedding ** 2, axis=0, keepdims=True).reshape(_K, 1)
    em2 = embedding * (-2.0)
    k = jnp.arange(_K, dtype=jnp.float32)
    g = jnp.concatenate(
        [embedding,
         jnp.ones((1, _K), jnp.float32),
         jnp.floor_divide(k, 8.0).reshape(1, _K),
         jnp.mod(k, 8.0).reshape(1, _K),
         jnp.zeros((_G - _C - 3, _K), jnp.float32)], axis=0)

    q, ind, dsum = pl.pallas_call(
        _vq_body,
        grid=(B // _BPB,),
        in_specs=[
            pl.BlockSpec((_BPB, C, S), lambda i: (i, 0, 0)),
            pl.BlockSpec((C, _K), lambda i: (0, 0)),
            pl.BlockSpec((_G, _K), lambda i: (0, 0)),
            pl.BlockSpec((_K, 1), lambda i: (0, 0)),
        ],
        out_specs=[
            pl.BlockSpec((_BPB, C, S), lambda i: (i, 0, 0)),
            pl.BlockSpec((_BPB, 1, S), lambda i: (i, 0, 0)),
            pl.BlockSpec((1, 1, 1), lambda i: (i, 0, 0)),
        ],
        out_shape=[
            jax.ShapeDtypeStruct((B, C, S), jnp.float32),
            jax.ShapeDtypeStruct((B, 1, S), jnp.int32),
            jax.ShapeDtypeStruct((B // _BPB, 1, 1), jnp.float32),
        ],
        compiler_params=pltpu.CompilerParams(
            dimension_semantics=("arbitrary",)),
    )(x, em2, g, b)

    quantize = q.reshape(B, C, H, W)
    embedding_ind = jnp.swapaxes(ind.reshape(B, H, W), 1, 2)
    diff = jnp.sum(dsum) * (0.25 / (B * S * C))
    return (quantize, diff, embedding_ind)


# speedup vs baseline: 1.0666x; 1.0123x over previous
import jax
import jax.numpy as jnp
from jax.experimental import pallas as pl
from jax.experimental.pallas import tpu as pltpu

_C = 64
_K = 1024
_S = 1024
_G = 72
_BPB = 16
_CS = 256


def _vq_body(x_ref, em2_ref, g_ref, b_ref, q_ref, ind_ref, dsum_ref):
    em2 = em2_ref[...]
    g = g_ref[...]
    b = b_ref[...]
    dsum = jnp.zeros((), jnp.float32)
    for j in range(_BPB):
        x = x_ref[j]
        a = jnp.sum(x * x, axis=0, keepdims=True)
        mm2 = jax.lax.dot_general(
            em2, x, (((0,), (0,)), ((), ())),
            preferred_element_type=jnp.float32)
        dist = (a + mm2) + b
        minv = jnp.min(dist, axis=0, keepdims=True)
        mask = dist == minv
        maskb = mask.astype(jnp.bfloat16)
        res = jax.lax.dot_general(
            g, maskb, (((1,), (0,)), ((), ())),
            preferred_element_type=jnp.float32)
        q_ref[j] = res[0:_C]
        ind_f = res[_C + 1:_C + 2] * 8.0 + res[_C + 2:_C + 3]
        ind_ref[j] = ind_f.astype(jnp.int32)
        dsum = dsum + jnp.sum(minv)

        cnt = res[_C:_C + 1]
        tie = jnp.max(cnt) > 1.5

        @pl.when(tie)
        def _():
            kio = jax.lax.broadcasted_iota(
                jnp.int32, (_K, _S), 0).astype(jnp.float32)
            t = jnp.where(mask, kio, jnp.float32(_K))
            ind2 = jnp.min(t, axis=0, keepdims=True)
            onehot = (t == ind2).astype(jnp.bfloat16)
            q_ref[j] = jax.lax.dot_general(
                g[0:_C], onehot, (((1,), (0,)), ((), ())),
                preferred_element_type=jnp.float32)
            ind_ref[j] = ind2.astype(jnp.int32)

    dsum_ref[...] = dsum.reshape(1, 1, 1)


def kernel(input, embedding):
    B, C, H, W = input.shape
    S = H * W
    x = input.reshape(B, C, S)
    b = jnp.sum(embedding ** 2, axis=0, keepdims=True).reshape(_K, 1)
    em2 = embedding * (-2.0)
    k = jnp.arange(_K, dtype=jnp.float32)
    g = jnp.concatenate(
        [embedding,
         jnp.ones((1, _K), jnp.float32),
         jnp.floor_divide(k, 8.0).reshape(1, _K),
         jnp.mod(k, 8.0).reshape(1, _K),
         jnp.zeros((_G - _C - 3, _K), jnp.float32)], axis=0)

    q, ind, dsum = pl.pallas_call(
        _vq_body,
        grid=(B // _BPB,),
        in_specs=[
            pl.BlockSpec((_BPB, C, S), lambda i: (i, 0, 0)),
            pl.BlockSpec((C, _K), lambda i: (0, 0)),
            pl.BlockSpec((_G, _K), lambda i: (0, 0)),
            pl.BlockSpec((_K, 1), lambda i: (0, 0)),
        ],
        out_specs=[
            pl.BlockSpec((_BPB, C, S), lambda i: (i, 0, 0)),
            pl.BlockSpec((_BPB, 1, S), lambda i: (i, 0, 0)),
            pl.BlockSpec((1, 1, 1), lambda i: (i, 0, 0)),
        ],
        out_shape=[
            jax.ShapeDtypeStruct((B, C, S), jnp.float32),
            jax.ShapeDtypeStruct((B, 1, S), jnp.int32),
            jax.ShapeDtypeStruct((B // _BPB, 1, 1), jnp.float32),
        ],
        compiler_params=pltpu.CompilerParams(
            dimension_semantics=("arbitrary",)),
    )(x, em2, g, b)

    quantize = q.reshape(B, C, H, W)
    embedding_ind = jnp.swapaxes(ind.reshape(B, H, W), 1, 2)
    diff = jnp.sum(dsum) * (0.25 / (B * S * C))
    return (quantize, diff, embedding_ind)


# speedup vs baseline: 1.0919x; 1.0237x over previous
import jax
import jax.numpy as jnp
from jax.experimental import pallas as pl
from jax.experimental.pallas import tpu as pltpu

_C = 64
_K = 1024
_S = 1024
_G = 72
_BPB = 4
_CS = 256


def _vq_body(x_ref, em2_ref, g_ref, b_ref, q_ref, ind_ref, dsum_ref):
    em2 = em2_ref[...]
    g = g_ref[...]
    b = b_ref[...]
    dsum = jnp.zeros((), jnp.float32)
    for j in range(_BPB):
        x = x_ref[j]
        a = jnp.sum(x * x, axis=0, keepdims=True)
        mm2 = jax.lax.dot_general(
            em2, x, (((0,), (0,)), ((), ())),
            preferred_element_type=jnp.float32)
        dist = (a + mm2) + b
        minv = jnp.min(dist, axis=0, keepdims=True)
        mask = dist == minv
        maskb = mask.astype(jnp.bfloat16)
        res = jax.lax.dot_general(
            g, maskb, (((1,), (0,)), ((), ())),
            preferred_element_type=jnp.float32)
        q_ref[j] = res[0:_C]
        ind_f = res[_C + 1:_C + 2] * 8.0 + res[_C + 2:_C + 3]
        ind_ref[j] = ind_f.astype(jnp.int32)
        dsum = dsum + jnp.sum(minv)

        cnt = res[_C:_C + 1]
        tie = jnp.max(cnt) > 1.5

        @pl.when(tie)
        def _():
            kio = jax.lax.broadcasted_iota(
                jnp.int32, (_K, _S), 0).astype(jnp.float32)
            t = jnp.where(mask, kio, jnp.float32(_K))
            ind2 = jnp.min(t, axis=0, keepdims=True)
            onehot = (t == ind2).astype(jnp.bfloat16)
            q_ref[j] = jax.lax.dot_general(
                g[0:_C], onehot, (((1,), (0,)), ((), ())),
                preferred_element_type=jnp.float32)
            ind_ref[j] = ind2.astype(jnp.int32)

    dsum_ref[...] = dsum.reshape(1, 1, 1)


def kernel(input, embedding):
    B, C, H, W = input.shape
    S = H * W
    x = input.reshape(B, C, S)
    b = jnp.sum(embedding ** 2, axis=0, keepdims=True).reshape(_K, 1)
    em2 = embedding * (-2.0)
    k = jnp.arange(_K, dtype=jnp.float32)
    g = jnp.concatenate(
        [embedding,
         jnp.ones((1, _K), jnp.float32),
         jnp.floor_divide(k, 8.0).reshape(1, _K),
         jnp.mod(k, 8.0).reshape(1, _K),
         jnp.zeros((_G - _C - 3, _K), jnp.float32)], axis=0)

    q, ind, dsum = pl.pallas_call(
        _vq_body,
        grid=(B // _BPB,),
        in_specs=[
            pl.BlockSpec((_BPB, C, S), lambda i: (i, 0, 0)),
            pl.BlockSpec((C, _K), lambda i: (0, 0)),
            pl.BlockSpec((_G, _K), lambda i: (0, 0)),
            pl.BlockSpec((_K, 1), lambda i: (0, 0)),
        ],
        out_specs=[
            pl.BlockSpec((_BPB, C, S), lambda i: (i, 0, 0)),
            pl.BlockSpec((_BPB, 1, S), lambda i: (i, 0, 0)),
            pl.BlockSpec((1, 1, 1), lambda i: (i, 0, 0)),
        ],
        out_shape=[
            jax.ShapeDtypeStruct((B, C, S), jnp.float32),
            jax.ShapeDtypeStruct((B, 1, S), jnp.int32),
            jax.ShapeDtypeStruct((B // _BPB, 1, 1), jnp.float32),
        ],
        compiler_params=pltpu.CompilerParams(
            dimension_semantics=("arbitrary",)),
    )(x, em2, g, b)

    quantize = q.reshape(B, C, H, W)
    embedding_ind = jnp.swapaxes(ind.reshape(B, H, W), 1, 2)
    diff = jnp.sum(dsum) * (0.25 / (B * S * C))
    return (quantize, diff, embedding_ind)
